# baseline (speedup 1.0000x reference)
import functools
import jax
import jax.numpy as jnp
from jax import lax
from jax.experimental import pallas as pl
from jax.experimental.pallas import tpu as pltpu
from jax.experimental.pallas import tpu_sc as plsc

N = 4096
D = 512
NW = 32
RPW = N // NW
CH = 32
NCH = RPW // CH
NJ = D // 16

BLK = 512
GRID = N // BLK


def _sc_body(sub_hbm, q_hbm, orig_hbm, out_sums_hbm, out_head_hbm,
             qv, sbuf, acc, tmp, sem_s):
    nc = 2
    c = lax.axis_index("c")
    s = lax.axis_index("s")
    wid = s * nc + c
    base = wid * RPW

    zero16 = jnp.zeros((16,), jnp.float32)
    for j in range(NJ):
        acc[0, pl.ds(j * 16, 16)] = zero16

    cps = [None] * NCH
    cps[0] = pltpu.async_copy(sub_hbm.at[pl.ds(base, CH)], sbuf.at[0], sem_s)

    for ci in range(NCH):
        pb = ci % 2
        if ci + 1 < NCH:
            cps[ci + 1] = pltpu.async_copy(
                sub_hbm.at[pl.ds(base + (ci + 1) * CH, CH)],
                sbuf.at[1 - pb], sem_s)
        cps[ci].wait()
        sb = sbuf.at[pb]

        def rbody(r, _, sb=sb):
            for j in range(NJ):
                v = sb[r, pl.ds(j * 16, 16)]
                plsc.addupdate(acc.at[0, pl.ds(j * 16, 16)], v)
            return _

        lax.fori_loop(0, CH, rbody, None)

    pltpu.sync_copy(acc, out_sums_hbm.at[wid])

    @pl.when(wid == 0)
    def _head():
        pltpu.sync_copy(q_hbm, qv)
        q = qv[pl.ds(0, 16)][0]
        pltpu.sync_copy(orig_hbm, tmp)
        pltpu.sync_copy(tmp, out_head_hbm.at[pl.ds(0, 1)])
        pltpu.sync_copy(sub_hbm.at[pl.ds(q, 1)], tmp)
        pltpu.sync_copy(tmp, out_head_hbm.at[pl.ds(1, 1)])


def _tc_body(sidx_ref, adj_ref, uni_ref, out_ref):
    g = pl.program_id(0)
    q = sidx_ref[0]

    @pl.when(g == 0)
    def _init():
        out_ref[...] = jnp.zeros((8, D), jnp.float32)

    lane = q % 128
    lane_ids = jax.lax.broadcasted_iota(jnp.int32, (BLK, 128), 1)
    colvals = jnp.sum(jnp.where(lane_ids == lane, adj_ref[...], 0.0), axis=1)
    maskf = (colvals > 0.0).astype(jnp.float32)

    row_ids = jax.lax.broadcasted_iota(jnp.int32, (8, BLK), 0)
    mat = jnp.where(row_ids == 0, maskf[None, :],
                    jnp.where(row_ids == 1, 1.0, 0.0))
    partial = jnp.dot(mat, uni_ref[...],
                      preferred_element_type=jnp.float32)
    out_ref[2:4, :] += partial[0:2, :]


def kernel(adj, cur_sub_idx, uni_feat, sub_feat, original_sub_feat):
    qarr = jnp.full((16,), cur_sub_idx, jnp.int32)
    orig = original_sub_feat.reshape((1, D))

    mesh = plsc.VectorSubcoreMesh(core_axis_name="c", subcore_axis_name="s")
    sc = functools.partial(
        pl.kernel,
        mesh=mesh,
        compiler_params=pltpu.CompilerParams(needs_layout_passes=False),
        out_type=(
            jax.ShapeDtypeStruct((NW, 1, D), jnp.float32),
            jax.ShapeDtypeStruct((2, D), jnp.float32),
        ),
        scratch_types=[
            pltpu.VMEM((16,), jnp.int32),
            pltpu.VMEM((2, CH, D), jnp.float32),
            pltpu.VMEM((1, D), jnp.float32),
            pltpu.VMEM((1, D), jnp.float32),
            pltpu.SemaphoreType.DMA,
        ],
    )(_sc_body)
    out_sums, out_head = sc(sub_feat, qarr, orig)

    sidx = jnp.asarray(cur_sub_idx, jnp.int32).reshape((1,))
    grid_spec = pltpu.PrefetchScalarGridSpec(
        num_scalar_prefetch=1,
        grid=(GRID,),
        in_specs=[
            pl.BlockSpec((BLK, 128), lambda g, s: (g, s[0] // 128)),
            pl.BlockSpec((BLK, D), lambda g, s: (g, 0)),
        ],
        out_specs=pl.BlockSpec((8, D), lambda g, s: (0, 0)),
    )
    tc_out = pl.pallas_call(
        _tc_body,
        grid_spec=grid_spec,
        out_shape=jax.ShapeDtypeStruct((8, D), jnp.float32),
    )(sidx, adj, uni_feat)

    sub_sum = jnp.sum(out_sums[:, 0, :], axis=0)
    return jnp.concatenate(
        (out_head[0], out_head[1], tc_out[2], tc_out[3], sub_sum))


# speedup vs baseline: 2.7456x; 2.7456x over previous
import jax
import jax.numpy as jnp
from jax.experimental import pallas as pl
from jax.experimental.pallas import tpu as pltpu

N = 4096
D = 512
BLK = 256
GRID = N // BLK


def _body(sidx_ref, adj_ref, uni_ref, sub_ref, orig_ref, out_ref):
    g = pl.program_id(0)
    q = sidx_ref[0]

    @pl.when(g == 0)
    def _init():
        out_ref[...] = jnp.zeros((8, D), jnp.float32)
        out_ref[0:1, :] = orig_ref[...]

    lane = q % 128
    lane_ids = jax.lax.broadcasted_iota(jnp.int32, (BLK, 128), 1)
    colvals = jnp.sum(jnp.where(lane_ids == lane, adj_ref[...], 0.0), axis=1,
                      keepdims=True)
    maskf = (colvals > 0.0).astype(jnp.float32)

    u = uni_ref[...]
    s = sub_ref[...]
    out_ref[2:3, :] += jnp.sum(u * maskf, axis=0, keepdims=True)
    out_ref[3:4, :] += jnp.sum(u, axis=0, keepdims=True)
    out_ref[4:5, :] += jnp.sum(s, axis=0, keepdims=True)

    @pl.when(g == q // BLK)
    def _cur():
        local = q - g * BLK
        row_ids = jax.lax.broadcasted_iota(jnp.int32, (BLK, D), 0)
        out_ref[1:2, :] = jnp.sum(jnp.where(row_ids == local, s, 0.0), axis=0,
                                  keepdims=True)


def kernel(adj, cur_sub_idx, uni_feat, sub_feat, original_sub_feat):
    sidx = jnp.asarray(cur_sub_idx, jnp.int32).reshape((1,))
    orig = original_sub_feat.reshape((1, D))
    grid_spec = pltpu.PrefetchScalarGridSpec(
        num_scalar_prefetch=1,
        grid=(GRID,),
        in_specs=[
            pl.BlockSpec((BLK, 128), lambda g, s: (g, s[0] // 128)),
            pl.BlockSpec((BLK, D), lambda g, s: (g, 0)),
            pl.BlockSpec((BLK, D), lambda g, s: (g, 0)),
            pl.BlockSpec((1, D), lambda g, s: (0, 0)),
        ],
        out_specs=pl.BlockSpec((8, D), lambda g, s: (0, 0)),
    )
    out = pl.pallas_call(
        _body,
        grid_spec=grid_spec,
        out_shape=jax.ShapeDtypeStruct((8, D), jnp.float32),
    )(sidx, adj, uni_feat, sub_feat, orig)
    return out[:5].reshape(-1)


# speedup vs baseline: 4.0027x; 1.4578x over previous
import jax
import jax.numpy as jnp
from jax.experimental import pallas as pl
from jax.experimental.pallas import tpu as pltpu

N = 4096
D = 512
BLK = 1024
GRID = N // BLK


def _body(sidx_ref, adj_ref, uni_ref, sub_ref, orig_ref, out_ref):
    g = pl.program_id(0)
    q = sidx_ref[0]

    @pl.when(g == 0)
    def _init():
        out_ref[...] = jnp.zeros((8, D), jnp.float32)
        out_ref[0:1, :] = orig_ref[...]

    lane = q % 128
    lane_ids = jax.lax.broadcasted_iota(jnp.int32, (BLK, 128), 1)
    colvals = jnp.sum(jnp.where(lane_ids == lane, adj_ref[...], 0.0), axis=1,
                      keepdims=True)
    maskf = (colvals > 0.0).astype(jnp.float32)

    u = uni_ref[...]
    s = sub_ref[...]
    out_ref[2:3, :] += jnp.sum(u * maskf, axis=0, keepdims=True)
    out_ref[3:4, :] += jnp.sum(u, axis=0, keepdims=True)
    out_ref[4:5, :] += jnp.sum(s, axis=0, keepdims=True)

    @pl.when(g == q // BLK)
    def _cur():
        local = q - g * BLK
        row_ids = jax.lax.broadcasted_iota(jnp.int32, (BLK, D), 0)
        out_ref[1:2, :] = jnp.sum(jnp.where(row_ids == local, s, 0.0), axis=0,
                                  keepdims=True)


def kernel(adj, cur_sub_idx, uni_feat, sub_feat, original_sub_feat):
    sidx = jnp.asarray(cur_sub_idx, jnp.int32).reshape((1,))
    orig = original_sub_feat.reshape((1, D))
    grid_spec = pltpu.PrefetchScalarGridSpec(
        num_scalar_prefetch=1,
        grid=(GRID,),
        in_specs=[
            pl.BlockSpec((BLK, 128), lambda g, s: (g, s[0] // 128)),
            pl.BlockSpec((BLK, D), lambda g, s: (g, 0)),
            pl.BlockSpec((BLK, D), lambda g, s: (g, 0)),
            pl.BlockSpec((1, D), lambda g, s: (0, 0)),
        ],
        out_specs=pl.BlockSpec((8, D), lambda g, s: (0, 0)),
    )
    out = pl.pallas_call(
        _body,
        grid_spec=grid_spec,
        out_shape=jax.ShapeDtypeStruct((8, D), jnp.float32),
    )(sidx, adj, uni_feat, sub_feat, orig)
    return out[:5].reshape(-1)


# speedup vs baseline: 4.2614x; 1.0646x over previous
import jax
import jax.numpy as jnp
from jax.experimental import pallas as pl
from jax.experimental.pallas import tpu as pltpu

N = 4096
D = 512
BLK = 2048
GRID = N // BLK


def _body(sidx_ref, adj_ref, uni_ref, sub_ref, orig_ref, out_ref):
    g = pl.program_id(0)
    q = sidx_ref[0]

    @pl.when(g == 0)
    def _init():
        out_ref[...] = jnp.zeros((8, D), jnp.float32)
        out_ref[0:1, :] = orig_ref[...]

    lane = q % 128
    lane_ids = jax.lax.broadcasted_iota(jnp.int32, (BLK, 128), 1)
    colvals = jnp.sum(jnp.where(lane_ids == lane, adj_ref[...], 0.0), axis=1,
                      keepdims=True)
    maskf = (colvals > 0.0).astype(jnp.float32)

    u = uni_ref[...]
    s = sub_ref[...]
    out_ref[2:3, :] += jnp.sum(u * maskf, axis=0, keepdims=True)
    out_ref[3:4, :] += jnp.sum(u, axis=0, keepdims=True)
    out_ref[4:5, :] += jnp.sum(s, axis=0, keepdims=True)

    @pl.when(g == q // BLK)
    def _cur():
        local = q - g * BLK
        row_ids = jax.lax.broadcasted_iota(jnp.int32, (BLK, D), 0)
        out_ref[1:2, :] = jnp.sum(jnp.where(row_ids == local, s, 0.0), axis=0,
                                  keepdims=True)


def kernel(adj, cur_sub_idx, uni_feat, sub_feat, original_sub_feat):
    sidx = jnp.asarray(cur_sub_idx, jnp.int32).reshape((1,))
    orig = original_sub_feat.reshape((1, D))
    grid_spec = pltpu.PrefetchScalarGridSpec(
        num_scalar_prefetch=1,
        grid=(GRID,),
        in_specs=[
            pl.BlockSpec((BLK, 128), lambda g, s: (g, s[0] // 128)),
            pl.BlockSpec((BLK, D), lambda g, s: (g, 0)),
            pl.BlockSpec((BLK, D), lambda g, s: (g, 0)),
            pl.BlockSpec((1, D), lambda g, s: (0, 0)),
        ],
        out_specs=pl.BlockSpec((8, D), lambda g, s: (0, 0)),
    )
    out = pl.pallas_call(
        _body,
        grid_spec=grid_spec,
        out_shape=jax.ShapeDtypeStruct((8, D), jnp.float32),
    )(sidx, adj, uni_feat, sub_feat, orig)
    return out[:5].reshape(-1)
